# baseline (speedup 1.0000x reference)
import jax
import jax.numpy as jnp
from jax import lax
from jax.experimental import pallas as pl
from jax.experimental.pallas import tpu as pltpu
from jax.experimental.pallas import tpu_sc as plsc

NPTS = 1000000
NH = 16384
E = NPTS * 3
NC, NS, L = 2, 16, 16
HALF_E = E // NC
HALF_ROWS = NPTS // NC

S = 12000
NCHUNK = HALF_E // S
NT = (NCHUNK + NS - 1) // NS

B = 128
KSH = NH // NS
KSH_ROWS = KSH // B
UQ_ROWS = KSH * 3 // B
SEG_CAP = 256
DUP_CAP = 2048
W_ROWS = (DUP_CAP // NS) * 3 // B
CNT_WORDS = 1 << 19


def _body(rest_hbm, hidx_hbm, hpts_hbm, delta_hbm, out_hbm,
          a0, a1, b0, b1, idx_flat, hash_share, c_share, ones_v,
          uq_dests, uq_gidx, uq_vals,
          segk_loc, segd_loc, n_loc, dup2d_k, dup2d_d, n_buf,
          dup_k, dup_d, w_dests, w_gidx, w_vals,
          sa0, sa1, sb0, sb1, so0, so1, sg,
          cnt, seg_k, seg_d, seg_n):
    core = lax.axis_index("c")
    sid = lax.axis_index("s")
    iota = lax.iota(jnp.int32, L)
    one16 = jnp.zeros((L,), jnp.int32) + 1
    zero16 = jnp.zeros((L,), jnp.int32)
    row_lo = core * HALF_ROWS
    row_hi = row_lo + HALF_ROWS

    def zero_body(i, _):
        idx_flat[pl.ds(i * L, L)] = zero16
        return 0
    lax.fori_loop(0, NH // L, zero_body, 0)
    for t in range(CNT_WORDS // NS // NH):
        pltpu.sync_copy(
            idx_flat,
            cnt.at[pl.ds(sid * (CNT_WORDS // NS) + t * NH, NH)])

    pltpu.sync_copy(hidx_hbm, idx_flat)
    k_base = sid * KSH
    for j in range(KSH_ROWS):
        for q in range(B // L):
            v = idx_flat[pl.ds(k_base + (j * (B // L) + q) * L, L)]
            hash_share[j, pl.ds(q * L, L)] = v >> 1
    def ones_body(i, _):
        ones_v[pl.ds(i * L, L)] = one16
        return 0
    lax.fori_loop(0, B // L, ones_body, 0)

    plsc.subcore_barrier()

    descs = [pltpu.async_copy(ones_v, cnt.at[hash_share.at[j]], sg, add=True)
             for j in range(KSH_ROWS)]
    for d in descs:
        d.wait()

    plsc.subcore_barrier()

    descs = [pltpu.async_copy(cnt.at[hash_share.at[j]], c_share.at[j], sg)
             for j in range(KSH_ROWS)]
    for d in descs:
        d.wait()

    n_uq = jnp.int32(0)
    n_sg = jnp.int32(0)
    for j in range(KSH_ROWS):
        for q in range(B // L):
            k0 = k_base + (j * (B // L) + q) * L
            d = idx_flat[pl.ds(k0, L)]
            c = c_share[j, pl.ds(q * L, L)]
            kv = k0 + iota
            flag = c >= 2
            m = (c == 1) & (d >= row_lo) & (d < row_hi)
            csum = jnp.cumsum(jnp.where(m, one16, zero16))
            pos = n_uq + csum - 1
            for t in range(3):
                e = pos * 3 + t
                plsc.store_scatter(uq_dests, [e >> 7, e & 127], d * 3 + t,
                                   mask=m)
                plsc.store_scatter(uq_gidx, [e >> 7, e & 127], kv * 3 + t,
                                   mask=m)
            n_uq = n_uq + csum[L - 1]
            csf = jnp.cumsum(jnp.where(flag, one16, zero16))
            sp = n_sg + csf - 1
            flag = flag & (sp < SEG_CAP)
            plsc.store_scatter(segk_loc, [sp], kv, mask=flag)
            plsc.store_scatter(segd_loc, [sp], d, mask=flag)
            n_sg = n_sg + csf[L - 1]
    n_sg = jnp.minimum(n_sg, SEG_CAP)

    @pl.when(n_uq > 0)
    def _pad_uq():
        d0 = uq_dests[0, pl.ds(0, L)][0]
        g0 = uq_gidx[0, pl.ds(0, L)][0]
        for r in range(UQ_ROWS):
            for q in range(B // L):
                i = r * (B // L) + q
                e = i * L + iota
                keep = e < n_uq * 3
                sl = pl.ds(q * L, L)
                uq_dests[r, sl] = jnp.where(keep, uq_dests[r, sl], d0)
                uq_gidx[r, sl] = jnp.where(keep, uq_gidx[r, sl], g0)

    n_loc[pl.ds(0, L)] = zero16 + n_sg
    pltpu.sync_copy(segk_loc, seg_k.at[sid])
    pltpu.sync_copy(segd_loc, seg_d.at[sid])
    pltpu.sync_copy(n_loc, seg_n.at[pl.ds(sid * L, L)])

    plsc.subcore_barrier()

    pltpu.sync_copy(seg_k, dup2d_k)
    pltpu.sync_copy(seg_d, dup2d_d)
    pltpu.sync_copy(seg_n, n_buf)

    n_dup = jnp.int32(0)
    for s in range(NS):
        ns = n_buf[pl.ds(s * L, L)][0]
        def merge_body(v, m_cur, s=s, ns=ns):
            kk = dup2d_k[s, pl.ds(v * L, L)]
            dd = dup2d_d[s, pl.ds(v * L, L)]
            m = (v * L + iota) < ns
            csum = jnp.cumsum(jnp.where(m, one16, zero16))
            pos = m_cur + csum - 1
            m = m & (pos < DUP_CAP)
            plsc.store_scatter(dup_k, [pos], kk, mask=m)
            plsc.store_scatter(dup_d, [pos], dd, mask=m)
            return m_cur + csum[L - 1]
        n_dup = lax.fori_loop(0, (ns + L - 1) // L, merge_body, n_dup)
    n_dup = jnp.minimum(n_dup, DUP_CAP)
    nv_dup = (n_dup + L - 1) // L

    def win_body(jj, w_cur):
        j = sid + jj * NS
        j_splat = zero16 + j
        dj_v = plsc.load_gather(dup_d, [j_splat])
        kj_v = plsc.load_gather(dup_k, [j_splat])
        def scan_body(v, acc):
            dd = dup_d[pl.ds(v * L, L)]
            kk = dup_k[pl.ds(v * L, L)]
            lanes = v * L + iota
            hit = (dd == dj_v) & (kk > kj_v) & (lanes < n_dup)
            return acc + jnp.where(hit, one16, zero16)
        acc = lax.fori_loop(0, nv_dup, scan_body, zero16)
        n_later = jnp.cumsum(acc)[L - 1]
        in_half = jnp.where((dj_v >= row_lo) & (dj_v < row_hi), one16, zero16)
        take = (n_later == 0) & (in_half[0] == 1)
        @pl.when(take)
        def _append():
            m3 = iota < 3
            e = w_cur * 3 + iota
            plsc.store_scatter(w_dests, [e >> 7, e & 127], dj_v * 3 + iota,
                               mask=m3)
            plsc.store_scatter(w_gidx, [e >> 7, e & 127], kj_v * 3 + iota,
                               mask=m3)
        return w_cur + jnp.where(take, jnp.int32(1), jnp.int32(0))
    n_j = jnp.maximum(jnp.int32(0), (n_dup - 1 - sid) // NS + 1)
    n_w = lax.fori_loop(0, n_j, win_body, jnp.int32(0))

    @pl.when(n_w > 0)
    def _pad_w():
        d0 = w_dests[0, pl.ds(0, L)][0]
        g0 = w_gidx[0, pl.ds(0, L)][0]
        for r in range(W_ROWS):
            for q in range(B // L):
                i = r * (B // L) + q
                e = i * L + iota
                keep = e < n_w * 3
                sl = pl.ds(q * L, L)
                w_dests[r, sl] = jnp.where(keep, w_dests[r, sl], d0)
                w_gidx[r, sl] = jnp.where(keep, w_gidx[r, sl], g0)

    @pl.when(n_uq > 0)
    def _gather_uq():
        ds_ = [pltpu.async_copy(hpts_hbm.at[uq_gidx.at[j]], uq_vals.at[j],
                                sg)
               for j in range(UQ_ROWS)]
        for d in ds_:
            d.wait()
    @pl.when(n_w > 0)
    def _gather_w():
        ds_ = [pltpu.async_copy(hpts_hbm.at[w_gidx.at[j]], w_vals.at[j], sg)
               for j in range(W_ROWS)]
        for d in ds_:
            d.wait()

    base = core * HALF_E
    abufs, bbufs = (a0, a1), (b0, b1)
    sas, sbs, sos = (sa0, sa1), (sb0, sb1), (so0, so1)

    def chunk_id(t):
        return sid + t * NS

    def start_in(t):
        cid = chunk_id(t)
        @pl.when(cid < NCHUNK)
        def _():
            off = base + cid * S
            pltpu.async_copy(rest_hbm.at[pl.ds(off, S)], abufs[t % 2],
                             sas[t % 2])
            pltpu.async_copy(delta_hbm.at[pl.ds(off, S)], bbufs[t % 2],
                             sbs[t % 2])

    def wait_in(t):
        @pl.when(chunk_id(t) < NCHUNK)
        def _():
            pltpu.make_async_copy(rest_hbm.at[pl.ds(0, S)], abufs[t % 2],
                                  sas[t % 2]).wait()
            pltpu.make_async_copy(delta_hbm.at[pl.ds(0, S)], bbufs[t % 2],
                                  sbs[t % 2]).wait()

    def wait_out(t):
        @pl.when(chunk_id(t) < NCHUNK)
        def _():
            pltpu.make_async_copy(abufs[t % 2], out_hbm.at[pl.ds(0, S)],
                                  sos[t % 2]).wait()

    start_in(0)
    for t in range(NT):
        wait_in(t)
        if t + 1 < NT:
            if t >= 1:
                wait_out(t - 1)
            start_in(t + 1)
        @pl.when(chunk_id(t) < NCHUNK)
        def _(t=t):
            a, b = abufs[t % 2], bbufs[t % 2]
            def add_body(i, _):
                sl = pl.ds(i * L, L)
                a[sl] = a[sl] + b[sl]
                return 0
            lax.fori_loop(0, S // L, add_body, 0)
            pltpu.async_copy(a, out_hbm.at[pl.ds(base + chunk_id(t) * S, S)],
                             sos[t % 2])
    wait_out(NT - 2)
    wait_out(NT - 1)

    plsc.subcore_barrier()

    @pl.when(n_uq > 0)
    def _scatter_uq():
        ds_ = [pltpu.async_copy(uq_vals.at[j], out_hbm.at[uq_dests.at[j]],
                                sg)
               for j in range(UQ_ROWS)]
        for d in ds_:
            d.wait()
    @pl.when(n_w > 0)
    def _scatter_w():
        ds_ = [pltpu.async_copy(w_vals.at[j], out_hbm.at[w_dests.at[j]], sg)
               for j in range(W_ROWS)]
        for d in ds_:
            d.wait()


@jax.jit
def _run(rest_f, hidx_flat, hpts_f, delta_f):
    mesh = plsc.VectorSubcoreMesh(
        core_axis_name="c", subcore_axis_name="s",
        num_cores=NC, num_subcores=NS)
    f = pl.kernel(
        _body,
        out_type=jax.ShapeDtypeStruct((E,), jnp.float32),
        mesh=mesh,
        compiler_params=pltpu.CompilerParams(needs_layout_passes=False),
        scratch_types=[
            pltpu.VMEM((S,), jnp.float32),
            pltpu.VMEM((S,), jnp.float32),
            pltpu.VMEM((S,), jnp.float32),
            pltpu.VMEM((S,), jnp.float32),
            pltpu.VMEM((NH,), jnp.int32),
            pltpu.VMEM((KSH_ROWS, B), jnp.int32),
            pltpu.VMEM((KSH_ROWS, B), jnp.int32),
            pltpu.VMEM((B,), jnp.int32),
            pltpu.VMEM((UQ_ROWS, B), jnp.int32),
            pltpu.VMEM((UQ_ROWS, B), jnp.int32),
            pltpu.VMEM((UQ_ROWS, B), jnp.float32),
            pltpu.VMEM((SEG_CAP,), jnp.int32),
            pltpu.VMEM((SEG_CAP,), jnp.int32),
            pltpu.VMEM((L,), jnp.int32),
            pltpu.VMEM((NS, SEG_CAP), jnp.int32),
            pltpu.VMEM((NS, SEG_CAP), jnp.int32),
            pltpu.VMEM((NS * L,), jnp.int32),
            pltpu.VMEM((DUP_CAP,), jnp.int32),
            pltpu.VMEM((DUP_CAP,), jnp.int32),
            pltpu.VMEM((W_ROWS, B), jnp.int32),
            pltpu.VMEM((W_ROWS, B), jnp.int32),
            pltpu.VMEM((W_ROWS, B), jnp.float32),
            pltpu.SemaphoreType.DMA,
            pltpu.SemaphoreType.DMA,
            pltpu.SemaphoreType.DMA,
            pltpu.SemaphoreType.DMA,
            pltpu.SemaphoreType.DMA,
            pltpu.SemaphoreType.DMA,
            pltpu.SemaphoreType.DMA,
            pltpu.VMEM_SHARED((CNT_WORDS,), jnp.int32),
            pltpu.VMEM_SHARED((NS, SEG_CAP), jnp.int32),
            pltpu.VMEM_SHARED((NS, SEG_CAP), jnp.int32),
            pltpu.VMEM_SHARED((NS * L,), jnp.int32),
        ],
    )
    return f(rest_f, hidx_flat, hpts_f, delta_f)


def kernel(rest_pts_tsr, handle_idx, handle_pts_tsr, delta_pts_tsr):
    rest_f = rest_pts_tsr.reshape(-1)
    delta_f = delta_pts_tsr.reshape(-1)
    hpts_f = handle_pts_tsr.reshape(-1)
    hidx_flat = handle_idx.astype(jnp.int32)
    out_f = _run(rest_f, hidx_flat, hpts_f, delta_f)
    return out_f.reshape(NPTS, 3)
